# baseline (speedup 1.0000x reference)
import functools

import jax
import jax.numpy as jnp
from jax import lax
from jax.experimental import pallas as pl
from jax.experimental.pallas import tpu as pltpu
from jax.experimental.pallas import tpu_sc as plsc

BS = 1024
NV = 501
NPAD = 512
CH = 128
D = 64
L = 16
NC = 2
NS = 16
NW = NC * NS
BPW = BS // NW
NCH = NPAD // CH
NT = BPW * NCH


def _trig_body(day_ref, w1_ref, w2_ref, dr_ref, di_ref):
    dayv = day_ref[:]
    dr_ref[:] = jnp.cos(w2_ref[:] * dayv)
    di_ref[:] = jnp.sin(w1_ref[:] * dayv)


def _trig(day, w1, w2):
    return pl.pallas_call(
        _trig_body,
        out_shape=(jax.ShapeDtypeStruct((BS, D), jnp.float32),
                   jax.ShapeDtypeStruct((BS, D), jnp.float32)),
    )(day.reshape(BS, 1), w1.reshape(1, D), w2.reshape(1, D))


def _loss_body(sc_ref, out_ref):
    s = sc_ref[:]
    col = lax.broadcasted_iota(jnp.int32, (BS, NPAD), 1)
    s = jnp.where(col < NV, s, -jnp.inf)
    m = jnp.max(s, axis=1, keepdims=True)
    e = jnp.exp(s - m)
    lse = jnp.log(jnp.sum(e, axis=1, keepdims=True)) + m
    loss2d = lse - sc_ref[:, 0:1]
    out_ref[:] = jnp.mean(loss2d).reshape(1, 1)


def _loss(scores):
    return pl.pallas_call(
        _loss_body,
        out_shape=jax.ShapeDtypeStruct((1, 1), jnp.float32),
    )(scores)


N_ENT_TBL = 1000000
TCH = 128
NFULL = N_ENT_TBL // TCH
CPW = NFULL // NW
NEXTRA = NFULL - CPW * NW

_tmesh = plsc.VectorSubcoreMesh(core_axis_name="c", subcore_axis_name="s")


@functools.partial(
    pl.kernel,
    mesh=_tmesh,
    compiler_params=pltpu.CompilerParams(needs_layout_passes=False),
    out_type=(jax.ShapeDtypeStruct((N_ENT_TBL * D,), jnp.float32),
              jax.ShapeDtypeStruct((N_ENT_TBL * D,), jnp.float32)),
    scratch_types=[
        pltpu.VMEM((D, TCH), jnp.float32),
        pltpu.VMEM((D, TCH), jnp.float32),
        pltpu.VMEM((TCH * D,), jnp.float32),
        pltpu.VMEM((TCH * D,), jnp.float32),
        pltpu.VMEM((D, 64), jnp.float32),
        pltpu.VMEM((64 * D,), jnp.float32),
        pltpu.SemaphoreType.DMA,
        pltpu.SemaphoreType.DMA,
        pltpu.SemaphoreType.DMA,
        pltpu.SemaphoreType.DMA,
    ],
)
def _tconv_sc(xt_r, xt_i, lin_r, lin_i, in_sl0, in_sl1, out_sl0, out_sl1,
              in_rem, out_rem, si0, si1, so0, so1):
    wid = lax.axis_index("s") * NC + lax.axis_index("c")
    c0w = wid * CPW
    sis = (si0, si1)
    sos = (so0, so1)
    ins = (in_sl0, in_sl1)
    outs = (out_sl0, out_sl1)
    segbase = [(seg * L + lax.iota(jnp.int32, L)) * D
               for seg in range(TCH // L)]

    for xt, lin in ((xt_r, lin_r), (xt_i, lin_i)):
        def fire_in(c, p):
            pltpu.async_copy(xt.at[:, pl.ds(c * TCH, TCH)], ins[p], sis[p])

        def wait_in(c, p):
            pltpu.make_async_copy(xt.at[:, pl.ds(c * TCH, TCH)],
                                  ins[p], sis[p]).wait()

        def fire_out(c, p):
            pltpu.async_copy(outs[p],
                             lin.at[pl.ds(c * TCH * D, TCH * D)], sos[p])

        def wait_out(c, p):
            pltpu.make_async_copy(outs[p],
                                  lin.at[pl.ds(c * TCH * D, TCH * D)],
                                  sos[p]).wait()

        def transpose_chunk(p):
            def tbody(d, carry):
                for seg in range(TCH // L):
                    vec = ins[p][d, pl.ds(seg * L, L)]
                    plsc.store_scatter(outs[p], [segbase[seg] + d], vec)
                return carry
            lax.fori_loop(0, D, tbody, 0)

        def tstep(i, p):
            c = c0w + i

            @pl.when(i + 1 < CPW)
            def _():
                fire_in(c + 1, 1 - p)

            wait_in(c, p)

            @pl.when(i >= 2)
            def _():
                wait_out(c - 2, p)

            transpose_chunk(p)
            fire_out(c, p)

        fire_in(jnp.int32(c0w), 0)

        def touter(ii, carry):
            tstep(2 * ii, 0)
            tstep(2 * ii + 1, 1)
            return carry
        lax.fori_loop(0, CPW // 2, touter, 0)
        wait_out(jnp.int32(c0w + CPW - 2), 0)
        wait_out(jnp.int32(c0w + CPW - 1), 1)

        @pl.when(wid < NEXTRA)
        def _():
            c = NW * CPW + wid
            pltpu.sync_copy(xt.at[:, pl.ds(c * TCH, TCH)], in_sl0)
            transpose_chunk(0)
            pltpu.sync_copy(out_sl0, lin.at[pl.ds(c * TCH * D, TCH * D)])

        @pl.when(wid == NEXTRA)
        def _():
            e0 = NFULL * TCH
            nrem = N_ENT_TBL - e0
            pltpu.sync_copy(xt.at[:, pl.ds(e0, nrem)], in_rem)

            def rbody(d, carry):
                for seg in range(nrem // L):
                    vec = in_rem[d, pl.ds(seg * L, L)]
                    plsc.store_scatter(out_rem, [segbase[seg] + d], vec)
                return carry
            lax.fori_loop(0, D, rbody, 0)
            pltpu.sync_copy(out_rem, lin.at[pl.ds(e0 * D, nrem * D)])


def _tconv_pair(emb_r, emb_i):
    lin_r, lin_i = _tconv_sc(emb_r.T, emb_i.T)
    n = emb_r.shape[0]
    return lin_r.reshape(n, D), lin_i.reshape(n, D)


_mesh = plsc.VectorSubcoreMesh(core_axis_name="c", subcore_axis_name="s")


@functools.partial(
    pl.kernel,
    mesh=_mesh,
    compiler_params=pltpu.CompilerParams(
        needs_layout_passes=False, use_tc_tiling_on_sc=False),
    out_type=jax.ShapeDtypeStruct((BS, NPAD), jnp.float32),
    scratch_types=[
        pltpu.VMEM((BPW, NCH, CH), jnp.int32),
        pltpu.VMEM((2, CH, D), jnp.float32),
        pltpu.VMEM((2, CH, D), jnp.float32),
        pltpu.VMEM((BPW,), jnp.int32),
        pltpu.VMEM((BPW,), jnp.int32),
        pltpu.VMEM((BPW, D), jnp.float32),
        pltpu.VMEM((BPW, D), jnp.float32),
        pltpu.VMEM((BPW, D), jnp.float32),
        pltpu.VMEM((BPW, D), jnp.float32),
        pltpu.VMEM((BPW, D), jnp.float32),
        pltpu.VMEM((BPW, D), jnp.float32),
        pltpu.VMEM((BPW, D), jnp.float32),
        pltpu.VMEM((BPW, D), jnp.float32),
        pltpu.VMEM((BPW, NPAD), jnp.float32),
        pltpu.SemaphoreType.DMA,
        pltpu.SemaphoreType.DMA,
        pltpu.SemaphoreType.DMA,
        pltpu.SemaphoreType.DMA,
        pltpu.SemaphoreType.DMA,
    ],
)
def _score(ids_hbm, sub_hbm, rel_hbm, dreal_hbm, dimg_hbm,
           embEr_hbm, embEi_hbm, embRr_hbm, embRi_hbm,
           out_hbm,
           ids_v, er_buf, ei_buf, sub_i, rel_i, sr, si, rr, ri, dr, di,
           ar_all, ai_all, scores_v,
           s_er0, s_ei0, s_er1, s_ei1, s_misc):
    wid = lax.axis_index("s") * NC + lax.axis_index("c")
    b0 = wid * BPW

    pltpu.sync_copy(ids_hbm.at[pl.ds(b0, BPW)], ids_v)
    pltpu.sync_copy(sub_hbm.at[pl.ds(b0, BPW)], sub_i)
    pltpu.sync_copy(rel_hbm.at[pl.ds(b0, BPW)], rel_i)
    pltpu.sync_copy(dreal_hbm.at[pl.ds(b0, BPW)], dr)
    pltpu.sync_copy(dimg_hbm.at[pl.ds(b0, BPW)], di)
    pltpu.async_copy(embEr_hbm.at[sub_i], sr, s_misc).wait()
    pltpu.async_copy(embEi_hbm.at[sub_i], si, s_misc).wait()
    pltpu.async_copy(embRr_hbm.at[rel_i], rr, s_misc).wait()
    pltpu.async_copy(embRi_hbm.at[rel_i], ri, s_misc).wait()

    def a_body(bl, carry):
        for g in range(D // L):
            slg = pl.ds(g * L, L)
            c = dr[bl, slg]
            s = di[bl, slg]
            svr = sr[bl, slg]
            svi = si[bl, slg]
            ar_all[bl, slg] = svr * c - svi * s + rr[bl, slg]
            ai_all[bl, slg] = svr * s + svi * c + ri[bl, slg]
        return carry
    lax.fori_loop(0, BPW, a_body, 0)

    sems = ((s_er0, s_ei0), (s_er1, s_ei1))

    def fire(t, p):
        bl = lax.div(t, NCH)
        ci = lax.rem(t, NCH)
        idx = ids_v.at[bl, ci]
        pltpu.async_copy(embEr_hbm.at[idx], er_buf.at[p], sems[p][0])
        pltpu.async_copy(embEi_hbm.at[idx], ei_buf.at[p], sems[p][1])

    def wait_for(t, p):
        bl = lax.div(t, NCH)
        ci = lax.rem(t, NCH)
        idx = ids_v.at[bl, ci]
        pltpu.make_async_copy(embEr_hbm.at[idx], er_buf.at[p], sems[p][0]).wait()
        pltpu.make_async_copy(embEi_hbm.at[idx], ei_buf.at[p], sems[p][1]).wait()

    fire(jnp.int32(0), 0)
    lane = lax.iota(jnp.int32, L)
    m0 = lane == 0

    def step(t, p):
        @pl.when(t + 1 < NT)
        def _():
            fire(t + 1, 1 - p)

        wait_for(t, p)
        bl = lax.div(t, NCH)
        ci = lax.rem(t, NCH)
        ebase = ci * CH
        bl_vec = jnp.broadcast_to(bl, (L,))
        cs = [dr[bl, pl.ds(g * L, L)] for g in range(D // L)]
        ss = [di[bl, pl.ds(g * L, L)] for g in range(D // L)]
        ars = [ar_all[bl, pl.ds(g * L, L)] for g in range(D // L)]
        ais = [ai_all[bl, pl.ds(g * L, L)] for g in range(D // L)]

        def e_body(j, carry):
            acc = jnp.zeros((L,), jnp.float32)
            for g in range(D // L):
                slg = pl.ds(g * L, L)
                er = er_buf[p, j, slg]
                ei = ei_buf[p, j, slg]
                vr = ars[g] - er * cs[g] + ei * ss[g]
                vi = ais[g] + er * ss[g] + ei * cs[g]
                acc = acc + jnp.abs(vr) + jnp.abs(vi)
            sv = jnp.broadcast_to(jnp.sum(acc), (L,))
            pos_vec = jnp.broadcast_to(ebase + j, (L,))
            plsc.store_scatter(scores_v, [bl_vec, pos_vec], sv, mask=m0)
            return carry
        lax.fori_loop(0, CH, e_body, 0, unroll=2)

    def outer(tt, carry):
        step(2 * tt, 0)
        step(2 * tt + 1, 1)
        return carry
    lax.fori_loop(0, NT // 2, outer, 0)

    pltpu.sync_copy(scores_v, out_hbm.at[pl.ds(b0, BPW)])


def kernel(sub, rel, obj, year, month, day, neg, emb_E_real, emb_E_img,
           emb_R_real, emb_R_img, w1, w2):
    dreal, dimg = _trig(day, w1, w2)
    ids = jnp.concatenate([obj[:, None], neg], axis=1)
    ids = jnp.pad(ids, ((0, 0), (0, NPAD - NV)))
    ids = ids.reshape(BS, NPAD // CH, CH)
    linEr, linEi = _tconv_pair(emb_E_real, emb_E_img)
    scores = _score(ids, sub, rel, dreal, dimg,
                    linEr, linEi, emb_R_real, emb_R_img)
    return _loss(scores)[0, 0]


# speedup vs baseline: 1.8868x; 1.8868x over previous
import functools

import jax
import jax.numpy as jnp
from jax import lax
from jax.experimental import pallas as pl
from jax.experimental.pallas import tpu as pltpu
from jax.experimental.pallas import tpu_sc as plsc

BS = 1024
NV = 501
NPAD = 512
CH = 128
D = 64
L = 16
NC = 2
NS = 16
NW = NC * NS
BPW = BS // NW
NCH = NPAD // CH
NT = BPW * NCH


def _trig_body(day_ref, w1_ref, w2_ref, dr_ref, di_ref):
    dayv = day_ref[:]
    dr_ref[:] = jnp.cos(w2_ref[:] * dayv)
    di_ref[:] = jnp.sin(w1_ref[:] * dayv)


def _trig(day, w1, w2):
    return pl.pallas_call(
        _trig_body,
        out_shape=(jax.ShapeDtypeStruct((BS, D), jnp.float32),
                   jax.ShapeDtypeStruct((BS, D), jnp.float32)),
    )(day.reshape(BS, 1), w1.reshape(1, D), w2.reshape(1, D))


def _loss_body(sc_ref, out_ref):
    s = sc_ref[:]
    col = lax.broadcasted_iota(jnp.int32, (BS, NPAD), 1)
    s = jnp.where(col < NV, s, -jnp.inf)
    m = jnp.max(s, axis=1, keepdims=True)
    e = jnp.exp(s - m)
    lse = jnp.log(jnp.sum(e, axis=1, keepdims=True)) + m
    loss2d = lse - sc_ref[:, 0:1]
    out_ref[:] = jnp.mean(loss2d).reshape(1, 1)


def _loss(scores):
    return pl.pallas_call(
        _loss_body,
        out_shape=jax.ShapeDtypeStruct((1, 1), jnp.float32),
    )(scores)


_mesh = plsc.VectorSubcoreMesh(core_axis_name="c", subcore_axis_name="s")


@functools.partial(
    pl.kernel,
    mesh=_mesh,
    compiler_params=pltpu.CompilerParams(
        needs_layout_passes=False, use_tc_tiling_on_sc=False),
    out_type=jax.ShapeDtypeStruct((BS, NPAD), jnp.float32),
    scratch_types=[
        pltpu.VMEM((BPW, NCH, CH), jnp.int32),
        pltpu.VMEM((2, CH, D), jnp.float32),
        pltpu.VMEM((2, CH, D), jnp.float32),
        pltpu.VMEM((BPW,), jnp.int32),
        pltpu.VMEM((BPW,), jnp.int32),
        pltpu.VMEM((BPW, D), jnp.float32),
        pltpu.VMEM((BPW, D), jnp.float32),
        pltpu.VMEM((BPW, D), jnp.float32),
        pltpu.VMEM((BPW, D), jnp.float32),
        pltpu.VMEM((BPW, D), jnp.float32),
        pltpu.VMEM((BPW, D), jnp.float32),
        pltpu.VMEM((BPW, D), jnp.float32),
        pltpu.VMEM((BPW, D), jnp.float32),
        pltpu.VMEM((BPW, NPAD), jnp.float32),
        pltpu.SemaphoreType.DMA,
        pltpu.SemaphoreType.DMA,
        pltpu.SemaphoreType.DMA,
        pltpu.SemaphoreType.DMA,
        pltpu.SemaphoreType.DMA,
    ],
)
def _score(ids_hbm, sub_hbm, rel_hbm, dreal_hbm, dimg_hbm,
           embEr_hbm, embEi_hbm, embRr_hbm, embRi_hbm,
           out_hbm,
           ids_v, er_buf, ei_buf, sub_i, rel_i, sr, si, rr, ri, dr, di,
           ar_all, ai_all, scores_v,
           s_er0, s_ei0, s_er1, s_ei1, s_misc):
    wid = lax.axis_index("s") * NC + lax.axis_index("c")
    b0 = wid * BPW

    pltpu.sync_copy(ids_hbm.at[pl.ds(b0, BPW)], ids_v)
    pltpu.sync_copy(sub_hbm.at[pl.ds(b0, BPW)], sub_i)
    pltpu.sync_copy(rel_hbm.at[pl.ds(b0, BPW)], rel_i)
    pltpu.sync_copy(dreal_hbm.at[pl.ds(b0, BPW)], dr)
    pltpu.sync_copy(dimg_hbm.at[pl.ds(b0, BPW)], di)
    pltpu.async_copy(embEr_hbm.at[sub_i], sr, s_misc).wait()
    pltpu.async_copy(embEi_hbm.at[sub_i], si, s_misc).wait()
    pltpu.async_copy(embRr_hbm.at[rel_i], rr, s_misc).wait()
    pltpu.async_copy(embRi_hbm.at[rel_i], ri, s_misc).wait()

    def a_body(bl, carry):
        for g in range(D // L):
            slg = pl.ds(g * L, L)
            c = dr[bl, slg]
            s = di[bl, slg]
            svr = sr[bl, slg]
            svi = si[bl, slg]
            ar_all[bl, slg] = svr * c - svi * s + rr[bl, slg]
            ai_all[bl, slg] = svr * s + svi * c + ri[bl, slg]
        return carry
    lax.fori_loop(0, BPW, a_body, 0)

    sems = ((s_er0, s_ei0), (s_er1, s_ei1))

    def fire(t, p):
        bl = lax.div(t, NCH)
        ci = lax.rem(t, NCH)
        idx = ids_v.at[bl, ci]
        pltpu.async_copy(embEr_hbm.at[idx], er_buf.at[p], sems[p][0])
        pltpu.async_copy(embEi_hbm.at[idx], ei_buf.at[p], sems[p][1])

    def wait_for(t, p):
        bl = lax.div(t, NCH)
        ci = lax.rem(t, NCH)
        idx = ids_v.at[bl, ci]
        pltpu.make_async_copy(embEr_hbm.at[idx], er_buf.at[p], sems[p][0]).wait()
        pltpu.make_async_copy(embEi_hbm.at[idx], ei_buf.at[p], sems[p][1]).wait()

    fire(jnp.int32(0), 0)
    lane = lax.iota(jnp.int32, L)
    m0 = lane == 0

    def step(t, p):
        @pl.when(t + 1 < NT)
        def _():
            fire(t + 1, 1 - p)

        wait_for(t, p)
        bl = lax.div(t, NCH)
        ci = lax.rem(t, NCH)
        ebase = ci * CH
        bl_vec = jnp.broadcast_to(bl, (L,))
        cs = [dr[bl, pl.ds(g * L, L)] for g in range(D // L)]
        ss = [di[bl, pl.ds(g * L, L)] for g in range(D // L)]
        ars = [ar_all[bl, pl.ds(g * L, L)] for g in range(D // L)]
        ais = [ai_all[bl, pl.ds(g * L, L)] for g in range(D // L)]

        def e_body(j, carry):
            acc = jnp.zeros((L,), jnp.float32)
            for g in range(D // L):
                slg = pl.ds(g * L, L)
                er = er_buf[p, j, slg]
                ei = ei_buf[p, j, slg]
                vr = ars[g] - er * cs[g] + ei * ss[g]
                vi = ais[g] + er * ss[g] + ei * cs[g]
                acc = acc + jnp.abs(vr) + jnp.abs(vi)
            sv = jnp.broadcast_to(jnp.sum(acc), (L,))
            pos_vec = jnp.broadcast_to(ebase + j, (L,))
            plsc.store_scatter(scores_v, [bl_vec, pos_vec], sv, mask=m0)
            return carry
        lax.fori_loop(0, CH, e_body, 0, unroll=2)

    def outer(tt, carry):
        step(2 * tt, 0)
        step(2 * tt + 1, 1)
        return carry
    lax.fori_loop(0, NT // 2, outer, 0)

    pltpu.sync_copy(scores_v, out_hbm.at[pl.ds(b0, BPW)])


def kernel(sub, rel, obj, year, month, day, neg, emb_E_real, emb_E_img,
           emb_R_real, emb_R_img, w1, w2):
    dreal, dimg = _trig(day, w1, w2)
    ids = jnp.concatenate([obj[:, None], neg], axis=1)
    ids = jnp.pad(ids, ((0, 0), (0, NPAD - NV)))
    ids = ids.reshape(BS, NPAD // CH, CH)
    scores = _score(ids, sub, rel, dreal, dimg,
                    emb_E_real, emb_E_img, emb_R_real, emb_R_img)
    return _loss(scores)[0, 0]


# speedup vs baseline: 1.8975x; 1.0056x over previous
import functools

import jax
import jax.numpy as jnp
from jax import lax
from jax.experimental import pallas as pl
from jax.experimental.pallas import tpu as pltpu
from jax.experimental.pallas import tpu_sc as plsc

BS = 1024
NV = 501
NPAD = 512
CH = 128
D = 64
L = 16
NC = 2
NS = 16
NW = NC * NS
BPW = BS // NW
NCH = NPAD // CH
NT = BPW * NCH


def _trig_body(day_ref, w1_ref, w2_ref, dr_ref, di_ref):
    dayv = day_ref[:]
    dr_ref[:] = jnp.cos(w2_ref[:] * dayv)
    di_ref[:] = jnp.sin(w1_ref[:] * dayv)


def _trig(day, w1, w2):
    return pl.pallas_call(
        _trig_body,
        out_shape=(jax.ShapeDtypeStruct((BS, D), jnp.float32),
                   jax.ShapeDtypeStruct((BS, D), jnp.float32)),
    )(day.reshape(BS, 1), w1.reshape(1, D), w2.reshape(1, D))


def _loss_body(sc_ref, out_ref):
    s = sc_ref[:]
    col = lax.broadcasted_iota(jnp.int32, (BS, NPAD), 1)
    s = jnp.where(col < NV, s, -jnp.inf)
    m = jnp.max(s, axis=1, keepdims=True)
    e = jnp.exp(s - m)
    lse = jnp.log(jnp.sum(e, axis=1, keepdims=True)) + m
    loss2d = lse - sc_ref[:, 0:1]
    out_ref[:] = jnp.mean(loss2d).reshape(1, 1)


def _loss(scores):
    return pl.pallas_call(
        _loss_body,
        out_shape=jax.ShapeDtypeStruct((1, 1), jnp.float32),
    )(scores)


_mesh = plsc.VectorSubcoreMesh(core_axis_name="c", subcore_axis_name="s")


@functools.partial(
    pl.kernel,
    mesh=_mesh,
    compiler_params=pltpu.CompilerParams(
        needs_layout_passes=False, use_tc_tiling_on_sc=False),
    out_type=jax.ShapeDtypeStruct((BS, NPAD), jnp.float32),
    scratch_types=[
        pltpu.VMEM((BPW, NCH, CH), jnp.int32),
        pltpu.VMEM((2, CH, D), jnp.float32),
        pltpu.VMEM((2, CH, D), jnp.float32),
        pltpu.VMEM((BPW,), jnp.int32),
        pltpu.VMEM((BPW,), jnp.int32),
        pltpu.VMEM((BPW, D), jnp.float32),
        pltpu.VMEM((BPW, D), jnp.float32),
        pltpu.VMEM((BPW, D), jnp.float32),
        pltpu.VMEM((BPW, D), jnp.float32),
        pltpu.VMEM((BPW, D), jnp.float32),
        pltpu.VMEM((BPW, D), jnp.float32),
        pltpu.VMEM((BPW, D), jnp.float32),
        pltpu.VMEM((BPW, D), jnp.float32),
        pltpu.VMEM((BPW, NPAD), jnp.float32),
        pltpu.SemaphoreType.DMA,
        pltpu.SemaphoreType.DMA,
        pltpu.SemaphoreType.DMA,
        pltpu.SemaphoreType.DMA,
        pltpu.SemaphoreType.DMA,
    ],
)
def _score(ids_hbm, sub_hbm, rel_hbm, dreal_hbm, dimg_hbm,
           embEr_hbm, embEi_hbm, embRr_hbm, embRi_hbm,
           out_hbm,
           ids_v, er_buf, ei_buf, sub_i, rel_i, sr, si, rr, ri, dr, di,
           ar_all, ai_all, scores_v,
           s_er0, s_ei0, s_er1, s_ei1, s_misc):
    wid = lax.axis_index("s") * NC + lax.axis_index("c")
    b0 = wid * BPW

    pltpu.sync_copy(ids_hbm.at[pl.ds(b0, BPW)], ids_v)
    pltpu.sync_copy(sub_hbm.at[pl.ds(b0, BPW)], sub_i)
    pltpu.sync_copy(rel_hbm.at[pl.ds(b0, BPW)], rel_i)
    pltpu.sync_copy(dreal_hbm.at[pl.ds(b0, BPW)], dr)
    pltpu.sync_copy(dimg_hbm.at[pl.ds(b0, BPW)], di)
    pltpu.async_copy(embEr_hbm.at[sub_i], sr, s_misc).wait()
    pltpu.async_copy(embEi_hbm.at[sub_i], si, s_misc).wait()
    pltpu.async_copy(embRr_hbm.at[rel_i], rr, s_misc).wait()
    pltpu.async_copy(embRi_hbm.at[rel_i], ri, s_misc).wait()

    def a_body(bl, carry):
        for g in range(D // L):
            slg = pl.ds(g * L, L)
            c = dr[bl, slg]
            s = di[bl, slg]
            svr = sr[bl, slg]
            svi = si[bl, slg]
            ar_all[bl, slg] = svr * c - svi * s + rr[bl, slg]
            ai_all[bl, slg] = svr * s + svi * c + ri[bl, slg]
        return carry
    lax.fori_loop(0, BPW, a_body, 0)

    sems = ((s_er0, s_ei0), (s_er1, s_ei1))

    def fire(t, p):
        bl = lax.div(t, NCH)
        ci = lax.rem(t, NCH)
        idx = ids_v.at[bl, ci]
        pltpu.async_copy(embEr_hbm.at[idx], er_buf.at[p], sems[p][0])
        pltpu.async_copy(embEi_hbm.at[idx], ei_buf.at[p], sems[p][1])

    def wait_for(t, p):
        bl = lax.div(t, NCH)
        ci = lax.rem(t, NCH)
        idx = ids_v.at[bl, ci]
        pltpu.make_async_copy(embEr_hbm.at[idx], er_buf.at[p], sems[p][0]).wait()
        pltpu.make_async_copy(embEi_hbm.at[idx], ei_buf.at[p], sems[p][1]).wait()

    fire(jnp.int32(0), 0)
    lane = lax.iota(jnp.int32, L)
    m0 = lane == 0

    def step(t, p):
        @pl.when(t + 1 < NT)
        def _():
            fire(t + 1, 1 - p)

        wait_for(t, p)
        bl = lax.div(t, NCH)
        ci = lax.rem(t, NCH)
        ebase = ci * CH
        bl_vec = jnp.broadcast_to(bl, (L,))
        cs = [dr[bl, pl.ds(g * L, L)] for g in range(D // L)]
        ss = [di[bl, pl.ds(g * L, L)] for g in range(D // L)]
        ars = [ar_all[bl, pl.ds(g * L, L)] for g in range(D // L)]
        ais = [ai_all[bl, pl.ds(g * L, L)] for g in range(D // L)]

        def e_body(j, carry):
            acc = jnp.zeros((L,), jnp.float32)
            for g in range(D // L):
                slg = pl.ds(g * L, L)
                er = er_buf[p, j, slg]
                ei = ei_buf[p, j, slg]
                vr = ars[g] - er * cs[g] + ei * ss[g]
                vi = ais[g] + er * ss[g] + ei * cs[g]
                acc = acc + jnp.abs(vr) + jnp.abs(vi)
            sv = jnp.broadcast_to(jnp.sum(acc), (L,))
            pos_vec = jnp.broadcast_to(ebase + j, (L,))
            plsc.store_scatter(scores_v, [bl_vec, pos_vec], sv, mask=m0)
            return carry
        lax.fori_loop(0, CH, e_body, 0, unroll=8)

    def outer(tt, carry):
        step(2 * tt, 0)
        step(2 * tt + 1, 1)
        return carry
    lax.fori_loop(0, NT // 2, outer, 0)

    pltpu.sync_copy(scores_v, out_hbm.at[pl.ds(b0, BPW)])


def kernel(sub, rel, obj, year, month, day, neg, emb_E_real, emb_E_img,
           emb_R_real, emb_R_img, w1, w2):
    dreal, dimg = _trig(day, w1, w2)
    ids = jnp.concatenate([obj[:, None], neg], axis=1)
    ids = jnp.pad(ids, ((0, 0), (0, NPAD - NV)))
    ids = ids.reshape(BS, NPAD // CH, CH)
    scores = _score(ids, sub, rel, dreal, dimg,
                    emb_E_real, emb_E_img, emb_R_real, emb_R_img)
    return _loss(scores)[0, 0]


# speedup vs baseline: 1.9017x; 1.0022x over previous
import functools

import jax
import jax.numpy as jnp
from jax import lax
from jax.experimental import pallas as pl
from jax.experimental.pallas import tpu as pltpu
from jax.experimental.pallas import tpu_sc as plsc

BS = 1024
NV = 501
NPAD = 512
CH = 128
D = 64
L = 16
NC = 2
NS = 16
NW = NC * NS
BPW = BS // NW
NCH = NPAD // CH
NT = BPW * NCH


def _trig_body(day_ref, w1_ref, w2_ref, dr_ref, di_ref):
    dayv = day_ref[:]
    dr_ref[:] = jnp.cos(w2_ref[:] * dayv)
    di_ref[:] = jnp.sin(w1_ref[:] * dayv)


def _trig(day, w1, w2):
    return pl.pallas_call(
        _trig_body,
        out_shape=(jax.ShapeDtypeStruct((BS, D), jnp.float32),
                   jax.ShapeDtypeStruct((BS, D), jnp.float32)),
    )(day.reshape(BS, 1), w1.reshape(1, D), w2.reshape(1, D))


def _loss_body(sc_ref, out_ref):
    s = sc_ref[:]
    col = lax.broadcasted_iota(jnp.int32, (BS, NPAD), 1)
    s = jnp.where(col < NV, s, -jnp.inf)
    m = jnp.max(s, axis=1, keepdims=True)
    e = jnp.exp(s - m)
    lse = jnp.log(jnp.sum(e, axis=1, keepdims=True)) + m
    loss2d = lse - sc_ref[:, 0:1]
    out_ref[:] = jnp.mean(loss2d).reshape(1, 1)


def _loss(scores):
    return pl.pallas_call(
        _loss_body,
        out_shape=jax.ShapeDtypeStruct((1, 1), jnp.float32),
    )(scores)


_mesh = plsc.VectorSubcoreMesh(core_axis_name="c", subcore_axis_name="s")


@functools.partial(
    pl.kernel,
    mesh=_mesh,
    compiler_params=pltpu.CompilerParams(
        needs_layout_passes=False, use_tc_tiling_on_sc=False),
    out_type=jax.ShapeDtypeStruct((BS, NPAD), jnp.float32),
    scratch_types=[
        pltpu.VMEM((BPW, NCH, CH), jnp.int32),
        pltpu.VMEM((2, CH, D), jnp.float32),
        pltpu.VMEM((2, CH, D), jnp.float32),
        pltpu.VMEM((BPW,), jnp.int32),
        pltpu.VMEM((BPW,), jnp.int32),
        pltpu.VMEM((BPW, D), jnp.float32),
        pltpu.VMEM((BPW, D), jnp.float32),
        pltpu.VMEM((BPW, D), jnp.float32),
        pltpu.VMEM((BPW, D), jnp.float32),
        pltpu.VMEM((BPW, D), jnp.float32),
        pltpu.VMEM((BPW, D), jnp.float32),
        pltpu.VMEM((BPW, D), jnp.float32),
        pltpu.VMEM((BPW, D), jnp.float32),
        pltpu.VMEM((BPW, NPAD), jnp.float32),
        pltpu.SemaphoreType.DMA,
        pltpu.SemaphoreType.DMA,
        pltpu.SemaphoreType.DMA,
        pltpu.SemaphoreType.DMA,
        pltpu.SemaphoreType.DMA,
    ],
)
def _score(ids_hbm, sub_hbm, rel_hbm, dreal_hbm, dimg_hbm,
           embEr_hbm, embEi_hbm, embRr_hbm, embRi_hbm,
           out_hbm,
           ids_v, er_buf, ei_buf, sub_i, rel_i, sr, si, rr, ri, dr, di,
           ar_all, ai_all, scores_v,
           s_er0, s_ei0, s_er1, s_ei1, s_misc):
    wid = lax.axis_index("s") * NC + lax.axis_index("c")
    b0 = wid * BPW

    pltpu.sync_copy(ids_hbm.at[pl.ds(b0, BPW)], ids_v)
    pltpu.sync_copy(sub_hbm.at[pl.ds(b0, BPW)], sub_i)
    pltpu.sync_copy(rel_hbm.at[pl.ds(b0, BPW)], rel_i)
    pltpu.sync_copy(dreal_hbm.at[pl.ds(b0, BPW)], dr)
    pltpu.sync_copy(dimg_hbm.at[pl.ds(b0, BPW)], di)
    pltpu.async_copy(embEr_hbm.at[sub_i], sr, s_misc).wait()
    pltpu.async_copy(embEi_hbm.at[sub_i], si, s_misc).wait()
    pltpu.async_copy(embRr_hbm.at[rel_i], rr, s_misc).wait()
    pltpu.async_copy(embRi_hbm.at[rel_i], ri, s_misc).wait()

    def a_body(bl, carry):
        for g in range(D // L):
            slg = pl.ds(g * L, L)
            c = dr[bl, slg]
            s = di[bl, slg]
            svr = sr[bl, slg]
            svi = si[bl, slg]
            ar_all[bl, slg] = svr * c - svi * s + rr[bl, slg]
            ai_all[bl, slg] = svr * s + svi * c + ri[bl, slg]
        return carry
    lax.fori_loop(0, BPW, a_body, 0)

    sems = ((s_er0, s_ei0), (s_er1, s_ei1))

    def fire(t, p):
        bl = lax.div(t, NCH)
        ci = lax.rem(t, NCH)
        idx = ids_v.at[bl, ci]
        pltpu.async_copy(embEr_hbm.at[idx], er_buf.at[p], sems[p][0])
        pltpu.async_copy(embEi_hbm.at[idx], ei_buf.at[p], sems[p][1])

    def wait_for(t, p):
        bl = lax.div(t, NCH)
        ci = lax.rem(t, NCH)
        idx = ids_v.at[bl, ci]
        pltpu.make_async_copy(embEr_hbm.at[idx], er_buf.at[p], sems[p][0]).wait()
        pltpu.make_async_copy(embEi_hbm.at[idx], ei_buf.at[p], sems[p][1]).wait()

    fire(jnp.int32(0), 0)
    lane = lax.iota(jnp.int32, L)
    m0 = lane == 0

    def step(t, p):
        @pl.when(t + 1 < NT)
        def _():
            fire(t + 1, 1 - p)

        wait_for(t, p)
        bl = lax.div(t, NCH)
        ci = lax.rem(t, NCH)
        ebase = ci * CH
        bl_vec = jnp.broadcast_to(bl, (L,))
        cs = [dr[bl, pl.ds(g * L, L)] for g in range(D // L)]
        ss = [di[bl, pl.ds(g * L, L)] for g in range(D // L)]
        ars = [ar_all[bl, pl.ds(g * L, L)] for g in range(D // L)]
        ais = [ai_all[bl, pl.ds(g * L, L)] for g in range(D // L)]

        def e_body(j, carry):
            acc = jnp.zeros((L,), jnp.float32)
            for g in range(D // L):
                slg = pl.ds(g * L, L)
                er = er_buf[p, j, slg]
                ei = ei_buf[p, j, slg]
                vr = ars[g] - er * cs[g] + ei * ss[g]
                vi = ais[g] + er * ss[g] + ei * cs[g]
                acc = acc + jnp.abs(vr) + jnp.abs(vi)
            sv = jnp.broadcast_to(jnp.sum(acc), (L,))
            pos_vec = jnp.broadcast_to(ebase + j, (L,))
            plsc.store_scatter(scores_v, [bl_vec, pos_vec], sv, mask=m0)
            return carry
        lax.fori_loop(0, CH, e_body, 0, unroll=16)

    def outer(tt, carry):
        step(2 * tt, 0)
        step(2 * tt + 1, 1)
        return carry
    lax.fori_loop(0, NT // 2, outer, 0)

    pltpu.sync_copy(scores_v, out_hbm.at[pl.ds(b0, BPW)])


def kernel(sub, rel, obj, year, month, day, neg, emb_E_real, emb_E_img,
           emb_R_real, emb_R_img, w1, w2):
    dreal, dimg = _trig(day, w1, w2)
    ids = jnp.concatenate([obj[:, None], neg], axis=1)
    ids = jnp.pad(ids, ((0, 0), (0, NPAD - NV)))
    ids = ids.reshape(BS, NPAD // CH, CH)
    scores = _score(ids, sub, rel, dreal, dimg,
                    emb_E_real, emb_E_img, emb_R_real, emb_R_img)
    return _loss(scores)[0, 0]
